# baseline (speedup 1.0000x reference)
import functools

import jax
import jax.numpy as jnp
from jax.experimental import pallas as pl
from jax.experimental.pallas import tpu as pltpu
from jax.experimental.pallas import tpu_sc as plsc

_HID = 128
_EPS = 1e-12
_GATHER_WINDOW = 128
_BB = 8


def _sc_gather(word_emb, ids_flat):
    n_tokens = ids_flat.shape[0]
    mesh = plsc.VectorSubcoreMesh(
        core_axis_name="core", subcore_axis_name="subcore"
    )

    @functools.partial(
        pl.kernel,
        out_type=jax.ShapeDtypeStruct((n_tokens, _HID), jnp.float32),
        mesh=mesh,
    )
    def gather_kernel(w_hbm, i_hbm, o_hbm):
        def body(i_vmem, o_vmem):
            pltpu.sync_copy(w_hbm.at[i_vmem.at[0]], o_vmem)

        pltpu.emit_pipeline(
            body,
            grid=(n_tokens // _GATHER_WINDOW,),
            in_specs=[
                pl.BlockSpec((1, _GATHER_WINDOW), lambda i: (0, i))
            ],
            out_specs=[
                pl.BlockSpec((_GATHER_WINDOW, _HID), lambda i: (i, 0))
            ],
            core_axis_name=("core", "subcore"),
            dimension_semantics=(pltpu.PARALLEL,),
        )(i_hbm, o_hbm)

    return gather_kernel(word_emb, ids_flat.reshape(1, n_tokens))


def _ln_body(g_ref, tt_ref, pos_ref, te_ref, gam_ref, bet_ref, o_ref):
    x = g_ref[...]
    tt = tt_ref[...]
    pos = pos_ref[...]
    t0 = te_ref[0]
    t1 = te_ref[1]
    bb, t, hid = x.shape
    ttf = jax.lax.broadcast_in_dim(
        tt.astype(jnp.float32), (bb, t, hid), (0, 1)
    )
    typ = t0 + ttf * (t1 - t0)
    x = x + pos[None] + typ
    mean = jnp.mean(x, axis=-1, keepdims=True)
    xc = x - mean
    var = jnp.mean(xc * xc, axis=-1, keepdims=True)
    y = xc * jax.lax.rsqrt(var + _EPS)
    o_ref[...] = y * gam_ref[0] + bet_ref[0]


def _tc_layernorm(gathered3, token_type_ids, pos_emb, type_emb, gamma2, beta2):
    b, t = token_type_ids.shape
    grid = (b // _BB,)
    return pl.pallas_call(
        _ln_body,
        grid=grid,
        in_specs=[
            pl.BlockSpec((_BB, t, _HID), lambda i: (i, 0, 0)),
            pl.BlockSpec((_BB, t), lambda i: (i, 0)),
            pl.BlockSpec((t, _HID), lambda i: (0, 0)),
            pl.BlockSpec((2, _HID), lambda i: (0, 0)),
            pl.BlockSpec((1, _HID), lambda i: (0, 0)),
            pl.BlockSpec((1, _HID), lambda i: (0, 0)),
        ],
        out_specs=pl.BlockSpec((_BB, t, _HID), lambda i: (i, 0, 0)),
        out_shape=jax.ShapeDtypeStruct((b, t, _HID), jnp.float32),
    )(gathered3, token_type_ids, pos_emb, type_emb, gamma2, beta2)


_N_CHUNKS = 8


@jax.jit
def kernel(input_ids, token_type_ids, word_emb, pos_emb, type_emb, ln_gamma, ln_beta):
    b, t = input_ids.shape
    ids_flat = input_ids.reshape(b * t).astype(jnp.int32)
    gamma2 = ln_gamma.reshape(1, _HID)
    beta2 = ln_beta.reshape(1, _HID)
    cb = b // _N_CHUNKS
    outs = []
    for k in range(_N_CHUNKS):
        ids_k = jax.lax.dynamic_slice_in_dim(ids_flat, k * cb * t, cb * t)
        tt_k = jax.lax.dynamic_slice_in_dim(token_type_ids, k * cb, cb)
        g = _sc_gather(word_emb, ids_k)
        outs.append(
            _tc_layernorm(
                g.reshape(cb, t, _HID), tt_k, pos_emb, type_emb, gamma2, beta2
            )
        )
    return jnp.concatenate(outs, axis=0)


# speedup vs baseline: 1.3413x; 1.3413x over previous
import functools

import jax
import jax.numpy as jnp
from jax.experimental import pallas as pl
from jax.experimental.pallas import tpu as pltpu
from jax.experimental.pallas import tpu_sc as plsc

_HID = 128
_EPS = 1e-12
_GATHER_WINDOW = 128
_BB = 8


def _sc_gather(word_emb, ids_flat):
    n_tokens = ids_flat.shape[0]
    mesh = plsc.VectorSubcoreMesh(
        core_axis_name="core", subcore_axis_name="subcore"
    )

    @functools.partial(
        pl.kernel,
        out_type=jax.ShapeDtypeStruct((n_tokens, _HID), jnp.float32),
        mesh=mesh,
    )
    def gather_kernel(w_hbm, i_hbm, o_hbm):
        def body(i_vmem, o_vmem):
            pltpu.sync_copy(w_hbm.at[i_vmem.at[0]], o_vmem)

        pltpu.emit_pipeline(
            body,
            grid=(n_tokens // _GATHER_WINDOW,),
            in_specs=[
                pl.BlockSpec((1, _GATHER_WINDOW), lambda i: (0, i))
            ],
            out_specs=[
                pl.BlockSpec((_GATHER_WINDOW, _HID), lambda i: (i, 0))
            ],
            core_axis_name=("core", "subcore"),
            dimension_semantics=(pltpu.PARALLEL,),
        )(i_hbm, o_hbm)

    return gather_kernel(word_emb, ids_flat.reshape(1, n_tokens))


def _ln_body(g_ref, tt_ref, pos_ref, te_ref, gam_ref, bet_ref, o_ref):
    x = g_ref[...]
    tt = tt_ref[...]
    pos = pos_ref[...]
    t0 = te_ref[0]
    t1 = te_ref[1]
    bb, t, hid = x.shape
    ttf = jax.lax.broadcast_in_dim(
        tt.astype(jnp.float32), (bb, t, hid), (0, 1)
    )
    typ = t0 + ttf * (t1 - t0)
    x = x + pos[None] + typ
    mean = jnp.mean(x, axis=-1, keepdims=True)
    xc = x - mean
    var = jnp.mean(xc * xc, axis=-1, keepdims=True)
    y = xc * jax.lax.rsqrt(var + _EPS)
    o_ref[...] = y * gam_ref[0] + bet_ref[0]


def _ln_body_acc(acc_ref, g_ref, tt_ref, pos_ref, te_ref, gam_ref, bet_ref, o_ref):
    del acc_ref
    _ln_body(g_ref, tt_ref, pos_ref, te_ref, gam_ref, bet_ref, o_ref)


def _tc_layernorm_into(acc, b_full, chunk_idx, gathered3, token_type_ids,
                       pos_emb, type_emb, gamma2, beta2):
    cb, t = token_type_ids.shape
    base = chunk_idx * (cb // _BB)
    grid = (cb // _BB,)
    chunk_specs = [
        pl.BlockSpec((_BB, t, _HID), lambda i: (i, 0, 0)),
        pl.BlockSpec((_BB, t), lambda i: (i, 0)),
        pl.BlockSpec((t, _HID), lambda i: (0, 0)),
        pl.BlockSpec((2, _HID), lambda i: (0, 0)),
        pl.BlockSpec((1, _HID), lambda i: (0, 0)),
        pl.BlockSpec((1, _HID), lambda i: (0, 0)),
    ]
    args = (gathered3, token_type_ids, pos_emb, type_emb, gamma2, beta2)
    out_spec = pl.BlockSpec((_BB, t, _HID), lambda i: (base + i, 0, 0))
    out_shape = jax.ShapeDtypeStruct((b_full, t, _HID), jnp.float32)
    if acc is None:
        return pl.pallas_call(
            _ln_body,
            grid=grid,
            in_specs=chunk_specs,
            out_specs=out_spec,
            out_shape=out_shape,
        )(*args)
    return pl.pallas_call(
        _ln_body_acc,
        grid=grid,
        in_specs=[pl.BlockSpec((1, 8, _HID), lambda i: (0, 0, 0))] + chunk_specs,
        out_specs=out_spec,
        out_shape=out_shape,
        input_output_aliases={0: 0},
    )(acc, *args)


_N_CHUNKS = 8


@jax.jit
def kernel(input_ids, token_type_ids, word_emb, pos_emb, type_emb, ln_gamma, ln_beta):
    b, t = input_ids.shape
    ids_flat = input_ids.reshape(b * t).astype(jnp.int32)
    gamma2 = ln_gamma.reshape(1, _HID)
    beta2 = ln_beta.reshape(1, _HID)
    cb = b // _N_CHUNKS
    acc = None
    for k in range(_N_CHUNKS):
        ids_k = jax.lax.dynamic_slice_in_dim(ids_flat, k * cb * t, cb * t)
        tt_k = jax.lax.dynamic_slice_in_dim(token_type_ids, k * cb, cb)
        g = _sc_gather(word_emb, ids_k)
        acc = _tc_layernorm_into(
            acc, b, k, g.reshape(cb, t, _HID), tt_k, pos_emb, type_emb,
            gamma2, beta2,
        )
    return acc
